# Initial kernel scaffold; baseline (speedup 1.0000x reference)
import jax
import jax.numpy as jnp
from jax.experimental import pallas as pl


def kernel(x, edge_index, edge_length, edge_attr, nn0_w, nn0_b, nn2_w, nn2_b, lin1_w, lin2_w, lin2_b, lin_w, lin_b):
    raise NotImplementedError("write your pallas kernel here")


# speedup vs baseline: 1.3006x; 1.3006x over previous
import functools

import jax
import jax.numpy as jnp
from jax import lax
from jax.experimental import pallas as pl
from jax.experimental.pallas import tpu as pltpu
from jax.experimental.pallas import tpu_sc as plsc

N, E, H, G = 10000, 320000, 128, 16
NP = 10240
HH = H // 2
NC, NS = 2, 16
EPW = E // NS
B = 80
NB = EPW // B
RPT = NP // NS
ZR = 128

_LOG2 = 0.6931471805599453
_PI = 3.141592653589793
_CUT = 10.0


def _ssp(v):
    return jnp.maximum(v, 0.0) + jnp.log1p(jnp.exp(-jnp.abs(v))) - _LOG2


def _filter_body(ea, el, w0, b0, w2, b2, out):
    a = ea[...]
    z = jnp.dot(a, w0[...], preferred_element_type=jnp.float32) + b0[...]
    w = jnp.dot(_ssp(z), w2[...], preferred_element_type=jnp.float32) + b2[...]
    l = el[...]
    env = 0.5 * (jnp.cos(l * (_PI / _CUT)) + 1.0)
    env = jnp.where((l <= _CUT) & (l >= 0.0), env, 0.0)
    w = w * env
    out[...] = jnp.stack([w[:, :HH], w[:, HH:]], axis=0)


def _edge_filter(edge_attr, edge_len2, w0t, b0, w2t, b2):
    be = 2560
    return pl.pallas_call(
        _filter_body,
        grid=(E // be,),
        in_specs=[
            pl.BlockSpec((be, G), lambda i: (i, 0)),
            pl.BlockSpec((be, 1), lambda i: (i, 0)),
            pl.BlockSpec((G, H), lambda i: (0, 0)),
            pl.BlockSpec((1, H), lambda i: (0, 0)),
            pl.BlockSpec((H, H), lambda i: (0, 0)),
            pl.BlockSpec((1, H), lambda i: (0, 0)),
        ],
        out_specs=pl.BlockSpec((NC, be, HH), lambda i: (0, i, 0)),
        out_shape=jax.ShapeDtypeStruct((NC, E, HH), jnp.float32),
    )(edge_attr, edge_len2, w0t, b0, w2t, b2)


def _lin1_body(xr, wr, out):
    h = jnp.dot(xr[...], wr[...], preferred_element_type=jnp.float32)
    out[pl.ds(0, N), :] = h[:, :HH]
    out[pl.ds(NP, N), :] = h[:, HH:]


def _lin1(x, w1t):
    return pl.pallas_call(
        _lin1_body,
        out_shape=jax.ShapeDtypeStruct((2 * NP, HH), jnp.float32),
    )(x, w1t)


def _tail_body(p, w2l, b2l, wl, bl, out):
    pv = p[...]
    r = jnp.concatenate([pv[0], pv[1]], axis=1)
    t = _ssp(jnp.dot(r, w2l[...], preferred_element_type=jnp.float32) + b2l[...])
    out[...] = jnp.dot(t, wl[...], preferred_element_type=jnp.float32) + bl[...]


def _tail(partial, w2lt, b2l, wlt, bl):
    bn = 2000
    return pl.pallas_call(
        _tail_body,
        grid=(N // bn,),
        in_specs=[
            pl.BlockSpec((NC, bn, HH), lambda i: (0, i, 0)),
            pl.BlockSpec((H, H), lambda i: (0, 0)),
            pl.BlockSpec((1, H), lambda i: (0, 0)),
            pl.BlockSpec((H, H), lambda i: (0, 0)),
            pl.BlockSpec((1, H), lambda i: (0, 0)),
        ],
        out_specs=pl.BlockSpec((bn, H), lambda i: (i, 0)),
        out_shape=jax.ShapeDtypeStruct((N, H), jnp.float32),
    )(partial, w2lt, b2l, wlt, bl)


@functools.cache
def _sc_msgpass_call():
    mesh = plsc.VectorSubcoreMesh(
        core_axis_name="c", subcore_axis_name="s",
        num_cores=NC, num_subcores=NS)
    return pl.kernel(
        _sc_msgpass,
        out_type=jax.ShapeDtypeStruct((NC, NP, HH), jnp.float32),
        mesh=mesh,
        scratch_types=[
            pltpu.VMEM((NB, B), jnp.int32),
            pltpu.VMEM((NB, B), jnp.int32),
            pltpu.VMEM((B, HH), jnp.float32),
            pltpu.VMEM((B, HH), jnp.float32),
            pltpu.VMEM((ZR, HH), jnp.float32),
            pltpu.VMEM_SHARED((NP, HH), jnp.float32),
            pltpu.SemaphoreType.DMA,
            pltpu.SemaphoreType.DMA,
        ],
        compiler_params=pltpu.CompilerParams(use_tc_tiling_on_sc=False),
    )


def _sc_msgpass(h_hbm, w_hbm, src_hbm, dst_hbm, out_hbm,
                src_v, dst_v, gbuf, wbuf, zbuf, acc, sem_g, sem_w):
    c = lax.axis_index("c")
    s = lax.axis_index("s")

    pltpu.sync_copy(src_hbm.at[s], src_v)
    pltpu.sync_copy(dst_hbm.at[s], dst_v)

    coff = jnp.full((16,), c * NP, jnp.int32)

    def orow(r, carry):
        for k in range(B // 16):
            sl = pl.ds(k * 16, 16)
            src_v[r, sl] = src_v[r, sl] + coff
        return carry

    lax.fori_loop(0, NB, orow, 0)

    zeros = jnp.zeros((16,), jnp.float32)

    def zrow(r, carry):
        for k in range(HH // 16):
            zbuf[r, pl.ds(k * 16, 16)] = zeros
        return carry

    lax.fori_loop(0, ZR, zrow, 0)

    def zcopy(t, carry):
        pltpu.sync_copy(zbuf, acc.at[pl.ds(s * RPT + t * ZR, ZR)])
        return carry

    lax.fori_loop(0, RPT // ZR, zcopy, 0)
    plsc.subcore_barrier()

    ebase = s * EPW

    def body(j, carry):
        gd = pltpu.async_copy(h_hbm.at[src_v.at[j]], gbuf, sem_g)
        wd = pltpu.async_copy(w_hbm.at[c, pl.ds(ebase + j * B, B)], wbuf, sem_w)
        gd.wait()
        wd.wait()

        def mrow(r, inner):
            for k in range(HH // 16):
                sl = pl.ds(k * 16, 16)
                gbuf[r, sl] = gbuf[r, sl] * wbuf[r, sl]
            return inner

        lax.fori_loop(0, B, mrow, 0)
        pltpu.sync_copy(gbuf, acc.at[dst_v.at[j]], add=True)
        return carry

    lax.fori_loop(0, NB, body, 0)

    plsc.subcore_barrier()
    pltpu.sync_copy(acc.at[pl.ds(s * RPT, RPT)],
                    out_hbm.at[c, pl.ds(s * RPT, RPT)])


def kernel(x, edge_index, edge_length, edge_attr,
           nn0_w, nn0_b, nn2_w, nn2_b, lin1_w, lin2_w, lin2_b, lin_w, lin_b):
    ei = edge_index.astype(jnp.int32)
    src3 = ei[0].reshape(NS, NB, B)
    dst3 = ei[1].reshape(NS, NB, B)
    el2 = edge_length.reshape(E, 1)

    w = _edge_filter(edge_attr, el2, nn0_w.T, nn0_b.reshape(1, H),
                     nn2_w.T, nn2_b.reshape(1, H))
    hcat = _lin1(x, lin1_w.T)
    partial = _sc_msgpass_call()(hcat, w, src3, dst3)
    return _tail(partial, lin2_w.T, lin2_b.reshape(1, H),
                 lin_w.T, lin_b.reshape(1, H))


# speedup vs baseline: 3.3683x; 2.5899x over previous
import functools

import jax
import jax.numpy as jnp
from jax import lax
from jax.experimental import pallas as pl
from jax.experimental.pallas import tpu as pltpu
from jax.experimental.pallas import tpu_sc as plsc

N, E, H, G = 10000, 320000, 128, 16
NP = 10240
HH = H // 2
NC, NS = 2, 16
EPW = E // NS
B = 80
NB = EPW // B
RPT = NP // NS
ZR = 128

_LOG2 = 0.6931471805599453
_PI = 3.141592653589793
_CUT = 10.0


def _ssp(v):
    return jnp.maximum(v, 0.0) + jnp.log1p(jnp.exp(-jnp.abs(v))) - _LOG2


_BE = 2560


def _filter_body(eat, el, w0, b0, w2, b2, out):
    a = eat[...]
    z = lax.dot_general(a, w0[...], (((0,), (0,)), ((), ())),
                        preferred_element_type=jnp.float32) + b0[...]
    w = jnp.dot(_ssp(z), w2[...], preferred_element_type=jnp.float32) + b2[...]
    l = el[0]
    env = 0.5 * (jnp.cos(l * (_PI / _CUT)) + 1.0)
    env = jnp.where((l <= _CUT) & (l >= 0.0), env, 0.0)
    w3 = w.reshape(_BE // 128, 128, H) * env[:, :, None]
    out[...] = w3.reshape(_BE, H)


def _edge_filter(edge_attr_t, el128, w0t, b0, w2t, b2):
    be = _BE
    return pl.pallas_call(
        _filter_body,
        grid=(E // be,),
        in_specs=[
            pl.BlockSpec((G, be), lambda i: (0, i)),
            pl.BlockSpec((1, be // 128, 128), lambda i: (i, 0, 0)),
            pl.BlockSpec((G, H), lambda i: (0, 0)),
            pl.BlockSpec((1, H), lambda i: (0, 0)),
            pl.BlockSpec((H, H), lambda i: (0, 0)),
            pl.BlockSpec((1, H), lambda i: (0, 0)),
        ],
        out_specs=pl.BlockSpec((be, H), lambda i: (i, 0)),
        out_shape=jax.ShapeDtypeStruct((E, H), jnp.float32),
    )(edge_attr_t, el128, w0t, b0, w2t, b2)


def _lin1_body(xr, wr, out):
    h = jnp.dot(xr[...], wr[...], preferred_element_type=jnp.float32)
    out[pl.ds(0, N), :] = h[:, :HH]
    out[pl.ds(NP, N), :] = h[:, HH:]


def _lin1(x, w1t):
    return pl.pallas_call(
        _lin1_body,
        out_shape=jax.ShapeDtypeStruct((2 * NP, HH), jnp.float32),
    )(x, w1t)


def _tail_body(p, w2l, b2l, wl, bl, out):
    pv = p[...]
    r = jnp.concatenate([pv[0], pv[1]], axis=1)
    t = _ssp(jnp.dot(r, w2l[...], preferred_element_type=jnp.float32) + b2l[...])
    out[...] = jnp.dot(t, wl[...], preferred_element_type=jnp.float32) + bl[...]


def _tail(partial, w2lt, b2l, wlt, bl):
    bn = 2000
    return pl.pallas_call(
        _tail_body,
        grid=(N // bn,),
        in_specs=[
            pl.BlockSpec((NC, bn, HH), lambda i: (0, i, 0)),
            pl.BlockSpec((H, H), lambda i: (0, 0)),
            pl.BlockSpec((1, H), lambda i: (0, 0)),
            pl.BlockSpec((H, H), lambda i: (0, 0)),
            pl.BlockSpec((1, H), lambda i: (0, 0)),
        ],
        out_specs=pl.BlockSpec((bn, H), lambda i: (i, 0)),
        out_shape=jax.ShapeDtypeStruct((N, H), jnp.float32),
    )(partial, w2lt, b2l, wlt, bl)


@functools.cache
def _sc_msgpass_call():
    mesh = plsc.VectorSubcoreMesh(
        core_axis_name="c", subcore_axis_name="s",
        num_cores=NC, num_subcores=NS)
    return pl.kernel(
        _sc_msgpass,
        out_type=jax.ShapeDtypeStruct((NC, NP, HH), jnp.float32),
        mesh=mesh,
        scratch_types=[
            pltpu.VMEM((NB, B), jnp.int32),
            pltpu.VMEM((NB, B), jnp.int32),
            pltpu.VMEM((B, HH), jnp.float32),
            pltpu.VMEM((B, HH), jnp.float32),
            pltpu.VMEM((ZR, HH), jnp.float32),
            pltpu.VMEM_SHARED((NP, HH), jnp.float32),
            pltpu.SemaphoreType.DMA,
            pltpu.SemaphoreType.DMA,
        ],
        compiler_params=pltpu.CompilerParams(use_tc_tiling_on_sc=False),
    )


def _sc_msgpass(h_hbm, w_hbm, src_hbm, dst_hbm, out_hbm,
                src_v, dst_v, gbuf, wbuf, zbuf, acc, sem_g, sem_w):
    c = lax.axis_index("c")
    s = lax.axis_index("s")

    pltpu.sync_copy(src_hbm.at[s], src_v)
    pltpu.sync_copy(dst_hbm.at[s], dst_v)

    coff = jnp.full((16,), c * NP, jnp.int32)

    def orow(r, carry):
        for k in range(B // 16):
            sl = pl.ds(k * 16, 16)
            src_v[r, sl] = src_v[r, sl] + coff
        return carry

    lax.fori_loop(0, NB, orow, 0)

    zeros = jnp.zeros((16,), jnp.float32)

    def zrow(r, carry):
        for k in range(HH // 16):
            zbuf[r, pl.ds(k * 16, 16)] = zeros
        return carry

    lax.fori_loop(0, ZR, zrow, 0)

    def zcopy(t, carry):
        pltpu.sync_copy(zbuf, acc.at[pl.ds(s * RPT + t * ZR, ZR)])
        return carry

    lax.fori_loop(0, RPT // ZR, zcopy, 0)
    plsc.subcore_barrier()

    ebase = s * EPW

    def body(j, carry):
        gd = pltpu.async_copy(h_hbm.at[src_v.at[j]], gbuf, sem_g)
        wd = pltpu.async_copy(
            w_hbm.at[pl.ds(ebase + j * B, B), pl.ds(c * HH, HH)], wbuf, sem_w)
        gd.wait()
        wd.wait()

        def mrow(r, inner):
            for k in range(HH // 16):
                sl = pl.ds(k * 16, 16)
                gbuf[r, sl] = gbuf[r, sl] * wbuf[r, sl]
            return inner

        lax.fori_loop(0, B, mrow, 0)
        pltpu.sync_copy(gbuf, acc.at[dst_v.at[j]], add=True)
        return carry

    lax.fori_loop(0, NB, body, 0)

    plsc.subcore_barrier()
    pltpu.sync_copy(acc.at[pl.ds(s * RPT, RPT)],
                    out_hbm.at[c, pl.ds(s * RPT, RPT)])


def kernel(x, edge_index, edge_length, edge_attr,
           nn0_w, nn0_b, nn2_w, nn2_b, lin1_w, lin2_w, lin2_b, lin_w, lin_b):
    ei = edge_index.astype(jnp.int32)
    src3 = ei[0].reshape(NS, NB, B)
    dst3 = ei[1].reshape(NS, NB, B)
    el128 = edge_length.reshape(E // _BE, _BE // 128, 128)

    w = _edge_filter(edge_attr.T, el128, nn0_w.T, nn0_b.reshape(1, H),
                     nn2_w.T, nn2_b.reshape(1, H))
    hcat = _lin1(x, lin1_w.T)
    partial = _sc_msgpass_call()(hcat, w, src3, dst3)
    return _tail(partial, lin2_w.T, lin2_b.reshape(1, H),
                 lin_w.T, lin_b.reshape(1, H))


# speedup vs baseline: 4.3925x; 1.3041x over previous
import functools

import jax
import jax.numpy as jnp
from jax import lax
from jax.experimental import pallas as pl
from jax.experimental.pallas import tpu as pltpu
from jax.experimental.pallas import tpu_sc as plsc

N, E, H, G = 10000, 320000, 128, 16
NP = 10240
HH = H // 2
NC, NS = 2, 16
EPW = E // NS
B = 80
NB = EPW // B
RPT = NP // NS
ZR = 128

_LOG2 = 0.6931471805599453
_PI = 3.141592653589793
_CUT = 10.0


def _ssp(v):
    return jnp.maximum(v, 0.0) + jnp.log1p(jnp.exp(-jnp.abs(v))) - _LOG2


_BE = 2560


def _filter_body(eat, el, w0, b0, w2, b2, out):
    a = eat[...]
    z = lax.dot_general(a, w0[...], (((0,), (0,)), ((), ())),
                        preferred_element_type=jnp.float32) + b0[...]
    w = jnp.dot(_ssp(z), w2[...], preferred_element_type=jnp.float32) + b2[...]
    l = el[0]
    env = 0.5 * (jnp.cos(l * (_PI / _CUT)) + 1.0)
    env = jnp.where((l <= _CUT) & (l >= 0.0), env, 0.0)
    w3 = w.reshape(_BE // 128, 128, H) * env[:, :, None]
    out[...] = w3.reshape(_BE, H)


def _edge_filter(edge_attr_t, el128, w0t, b0, w2t, b2):
    be = _BE
    return pl.pallas_call(
        _filter_body,
        grid=(E // be,),
        in_specs=[
            pl.BlockSpec((G, be), lambda i: (0, i)),
            pl.BlockSpec((1, be // 128, 128), lambda i: (i, 0, 0)),
            pl.BlockSpec((G, H), lambda i: (0, 0)),
            pl.BlockSpec((1, H), lambda i: (0, 0)),
            pl.BlockSpec((H, H), lambda i: (0, 0)),
            pl.BlockSpec((1, H), lambda i: (0, 0)),
        ],
        out_specs=pl.BlockSpec((be, H), lambda i: (i, 0)),
        out_shape=jax.ShapeDtypeStruct((E, H), jnp.float32),
    )(edge_attr_t, el128, w0t, b0, w2t, b2)


def _lin1_body(xr, wr, out):
    h = jnp.dot(xr[...], wr[...], preferred_element_type=jnp.float32)
    out[pl.ds(0, N), :] = h[:, :HH]
    out[pl.ds(NP, N), :] = h[:, HH:]


def _lin1(x, w1t):
    return pl.pallas_call(
        _lin1_body,
        out_shape=jax.ShapeDtypeStruct((2 * NP, HH), jnp.float32),
    )(x, w1t)


def _tail_body(p, w2l, b2l, wl, bl, out):
    pv = p[...]
    r = jnp.concatenate([pv[0], pv[1]], axis=1)
    t = _ssp(jnp.dot(r, w2l[...], preferred_element_type=jnp.float32) + b2l[...])
    out[...] = jnp.dot(t, wl[...], preferred_element_type=jnp.float32) + bl[...]


def _tail(partial, w2lt, b2l, wlt, bl):
    bn = 2000
    return pl.pallas_call(
        _tail_body,
        grid=(N // bn,),
        in_specs=[
            pl.BlockSpec((NC, bn, HH), lambda i: (0, i, 0)),
            pl.BlockSpec((H, H), lambda i: (0, 0)),
            pl.BlockSpec((1, H), lambda i: (0, 0)),
            pl.BlockSpec((H, H), lambda i: (0, 0)),
            pl.BlockSpec((1, H), lambda i: (0, 0)),
        ],
        out_specs=pl.BlockSpec((bn, H), lambda i: (i, 0)),
        out_shape=jax.ShapeDtypeStruct((N, H), jnp.float32),
    )(partial, w2lt, b2l, wlt, bl)


@functools.cache
def _sc_msgpass_call():
    mesh = plsc.VectorSubcoreMesh(
        core_axis_name="c", subcore_axis_name="s",
        num_cores=NC, num_subcores=NS)
    return pl.kernel(
        _sc_msgpass,
        out_type=jax.ShapeDtypeStruct((NC, NP, HH), jnp.float32),
        mesh=mesh,
        scratch_types=[
            pltpu.VMEM((NB, B), jnp.int32),
            pltpu.VMEM((NB, B), jnp.int32),
            pltpu.VMEM((B, HH), jnp.float32),
            pltpu.VMEM((B, HH), jnp.float32),
            pltpu.VMEM((B, HH), jnp.float32),
            pltpu.VMEM((B, HH), jnp.float32),
            pltpu.VMEM((ZR, HH), jnp.float32),
            pltpu.VMEM_SHARED((NP, HH), jnp.float32),
            pltpu.SemaphoreType.DMA,
            pltpu.SemaphoreType.DMA,
            pltpu.SemaphoreType.DMA,
            pltpu.SemaphoreType.DMA,
            pltpu.SemaphoreType.DMA,
            pltpu.SemaphoreType.DMA,
        ],
        compiler_params=pltpu.CompilerParams(use_tc_tiling_on_sc=False),
    )


def _sc_msgpass(h_hbm, w_hbm, src_hbm, dst_hbm, out_hbm,
                src_v, dst_v, gbuf_a, gbuf_b, wbuf_a, wbuf_b, zbuf, acc,
                sem_ga, sem_gb, sem_wa, sem_wb, sem_sa, sem_sb):
    c = lax.axis_index("c")
    s = lax.axis_index("s")

    pltpu.sync_copy(src_hbm.at[s], src_v)
    pltpu.sync_copy(dst_hbm.at[s], dst_v)

    coff = jnp.full((16,), c * NP, jnp.int32)

    def orow(r, carry):
        for k in range(B // 16):
            sl = pl.ds(k * 16, 16)
            src_v[r, sl] = src_v[r, sl] + coff
        return carry

    lax.fori_loop(0, NB, orow, 0)

    zeros = jnp.zeros((16,), jnp.float32)

    def zrow(r, carry):
        for k in range(HH // 16):
            zbuf[r, pl.ds(k * 16, 16)] = zeros
        return carry

    lax.fori_loop(0, ZR, zrow, 0)

    def zcopy(t, carry):
        pltpu.sync_copy(zbuf, acc.at[pl.ds(s * RPT + t * ZR, ZR)])
        return carry

    lax.fori_loop(0, RPT // ZR, zcopy, 0)
    plsc.subcore_barrier()

    ebase = s * EPW

    def fetch(j, gbuf, wbuf, sem_g, sem_w):
        pltpu.async_copy(h_hbm.at[src_v.at[j]], gbuf, sem_g)
        pltpu.async_copy(
            w_hbm.at[pl.ds(ebase + j * B, B), pl.ds(c * HH, HH)], wbuf, sem_w)

    def wait_fetch(j, gbuf, wbuf, sem_g, sem_w):
        pltpu.make_async_copy(h_hbm.at[src_v.at[j]], gbuf, sem_g).wait()
        pltpu.make_async_copy(
            w_hbm.at[pl.ds(ebase + j * B, B), pl.ds(c * HH, HH)],
            wbuf, sem_w).wait()

    def mul(gbuf, wbuf):
        def mrow(r4, inner):
            for dr in range(4):
                for k in range(HH // 16):
                    sl = pl.ds(k * 16, 16)
                    r = r4 * 4 + dr
                    gbuf[r, sl] = gbuf[r, sl] * wbuf[r, sl]
            return inner
        lax.fori_loop(0, B // 4, mrow, 0)

    def scat(j, gbuf, sem_s):
        pltpu.async_copy(gbuf, acc.at[dst_v.at[j]], sem_s, add=True)

    def wait_scat(j, gbuf, sem_s):
        pltpu.make_async_copy(gbuf, acc.at[dst_v.at[j]], sem_s).wait()

    fetch(0, gbuf_a, wbuf_a, sem_ga, sem_wa)

    def body(j2, carry):
        a = j2 * 2
        b = a + 1
        wait_fetch(a, gbuf_a, wbuf_a, sem_ga, sem_wa)

        @pl.when(j2 > 0)
        def _():
            wait_scat(b, gbuf_b, sem_sb)

        fetch(b, gbuf_b, wbuf_b, sem_gb, sem_wb)
        mul(gbuf_a, wbuf_a)
        scat(a, gbuf_a, sem_sa)
        wait_fetch(b, gbuf_b, wbuf_b, sem_gb, sem_wb)
        wait_scat(a, gbuf_a, sem_sa)

        @pl.when(j2 < NB // 2 - 1)
        def _():
            fetch(a + 2, gbuf_a, wbuf_a, sem_ga, sem_wa)

        mul(gbuf_b, wbuf_b)
        scat(b, gbuf_b, sem_sb)
        return carry

    lax.fori_loop(0, NB // 2, body, 0)
    wait_scat(NB - 1, gbuf_b, sem_sb)

    plsc.subcore_barrier()
    pltpu.sync_copy(acc.at[pl.ds(s * RPT, RPT)],
                    out_hbm.at[c, pl.ds(s * RPT, RPT)])


def kernel(x, edge_index, edge_length, edge_attr,
           nn0_w, nn0_b, nn2_w, nn2_b, lin1_w, lin2_w, lin2_b, lin_w, lin_b):
    ei = edge_index.astype(jnp.int32)
    src3 = ei[0].reshape(NS, NB, B)
    dst3 = ei[1].reshape(NS, NB, B)
    el128 = edge_length.reshape(E // _BE, _BE // 128, 128)

    w = _edge_filter(edge_attr.T, el128, nn0_w.T, nn0_b.reshape(1, H),
                     nn2_w.T, nn2_b.reshape(1, H))
    hcat = _lin1(x, lin1_w.T)
    partial = _sc_msgpass_call()(hcat, w, src3, dst3)
    return _tail(partial, lin2_w.T, lin2_b.reshape(1, H),
                 lin_w.T, lin_b.reshape(1, H))


# speedup vs baseline: 4.9494x; 1.1268x over previous
import functools

import jax
import jax.numpy as jnp
from jax import lax
from jax.experimental import pallas as pl
from jax.experimental.pallas import tpu as pltpu
from jax.experimental.pallas import tpu_sc as plsc

N, E, H, G = 10000, 320000, 128, 16
NP = 10240
HH = H // 2
NC, NS = 2, 16
EPW = E // NS
B = 80
NB = EPW // B
RPT = NP // NS
ZR = 128

_LOG2 = 0.6931471805599453
_PI = 3.141592653589793
_CUT = 10.0


def _ssp(v):
    return jnp.maximum(v, 0.0) + jnp.log1p(jnp.exp(-jnp.abs(v))) - _LOG2


_BE = 2560


def _filter_body(eat, el, w0, b0, w2, b2, out):
    a = eat[...]
    z = lax.dot_general(a, w0[...], (((0,), (0,)), ((), ())),
                        preferred_element_type=jnp.float32) + b0[...]
    w = jnp.dot(_ssp(z), w2[...], preferred_element_type=jnp.float32) + b2[...]
    l = el[0]
    env = 0.5 * (jnp.cos(l * (_PI / _CUT)) + 1.0)
    env = jnp.where((l <= _CUT) & (l >= 0.0), env, 0.0)
    w3 = w.reshape(_BE // 128, 128, H) * env[:, :, None]
    out[...] = w3.reshape(_BE, H)


def _edge_filter(edge_attr_t, el128, w0t, b0, w2t, b2, nblk, blk_off):
    be = _BE
    return pl.pallas_call(
        _filter_body,
        grid=(nblk,),
        in_specs=[
            pl.BlockSpec((G, be), lambda i: (0, blk_off + i)),
            pl.BlockSpec((1, be // 128, 128), lambda i: (blk_off + i, 0, 0)),
            pl.BlockSpec((G, H), lambda i: (0, 0)),
            pl.BlockSpec((1, H), lambda i: (0, 0)),
            pl.BlockSpec((H, H), lambda i: (0, 0)),
            pl.BlockSpec((1, H), lambda i: (0, 0)),
        ],
        out_specs=pl.BlockSpec((be, H), lambda i: (i, 0)),
        out_shape=jax.ShapeDtypeStruct((nblk * be, H), jnp.float32),
    )(edge_attr_t, el128, w0t, b0, w2t, b2)


def _lin1_body(xr, wr, out):
    h = jnp.dot(xr[...], wr[...], preferred_element_type=jnp.float32)
    out[pl.ds(0, N), :] = h[:, :HH]
    out[pl.ds(NP, N), :] = h[:, HH:]


def _lin1(x, w1t):
    return pl.pallas_call(
        _lin1_body,
        out_shape=jax.ShapeDtypeStruct((2 * NP, HH), jnp.float32),
    )(x, w1t)


def _tail_body(p0, p1, w2l, b2l, wl, bl, out):
    a = p0[...]
    b = p1[...]
    r = jnp.concatenate([a[0] + b[0], a[1] + b[1]], axis=1)
    t = _ssp(jnp.dot(r, w2l[...], preferred_element_type=jnp.float32) + b2l[...])
    out[...] = jnp.dot(t, wl[...], preferred_element_type=jnp.float32) + bl[...]


def _tail(part0, part1, w2lt, b2l, wlt, bl):
    bn = 2000
    return pl.pallas_call(
        _tail_body,
        grid=(N // bn,),
        in_specs=[
            pl.BlockSpec((NC, bn, HH), lambda i: (0, i, 0)),
            pl.BlockSpec((NC, bn, HH), lambda i: (0, i, 0)),
            pl.BlockSpec((H, H), lambda i: (0, 0)),
            pl.BlockSpec((1, H), lambda i: (0, 0)),
            pl.BlockSpec((H, H), lambda i: (0, 0)),
            pl.BlockSpec((1, H), lambda i: (0, 0)),
        ],
        out_specs=pl.BlockSpec((bn, H), lambda i: (i, 0)),
        out_shape=jax.ShapeDtypeStruct((N, H), jnp.float32),
    )(part0, part1, w2lt, b2l, wlt, bl)


@functools.cache
def _sc_msgpass_call(nb, chunk_base):
    epw = nb * B

    def _sc_msgpass(h_hbm, w_hbm, src_hbm, dst_hbm, out_hbm,
                    src_v, dst_v, gbuf_a, gbuf_b, wbuf_a, wbuf_b, zbuf, acc,
                    sem_ga, sem_gb, sem_wa, sem_wb, sem_sa, sem_sb):
        c = lax.axis_index("c")
        s = lax.axis_index("s")

        pltpu.sync_copy(src_hbm.at[s], src_v)
        pltpu.sync_copy(dst_hbm.at[s], dst_v)

        coff = jnp.full((16,), c * NP, jnp.int32)

        def orow(r, carry):
            for k in range(B // 16):
                sl = pl.ds(k * 16, 16)
                src_v[r, sl] = src_v[r, sl] + coff
            return carry

        lax.fori_loop(0, nb, orow, 0)

        zeros = jnp.zeros((16,), jnp.float32)

        def zrow(r, carry):
            for k in range(HH // 16):
                zbuf[r, pl.ds(k * 16, 16)] = zeros
            return carry

        lax.fori_loop(0, ZR, zrow, 0)

        def zcopy(t, carry):
            pltpu.sync_copy(zbuf, acc.at[pl.ds(s * RPT + t * ZR, ZR)])
            return carry

        lax.fori_loop(0, RPT // ZR, zcopy, 0)
        plsc.subcore_barrier()

        ebase = chunk_base + s * epw

        def fetch(j, gbuf, wbuf, sem_g, sem_w):
            pltpu.async_copy(h_hbm.at[src_v.at[j]], gbuf, sem_g)
            pltpu.async_copy(
                w_hbm.at[pl.ds(ebase + j * B, B), pl.ds(c * HH, HH)],
                wbuf, sem_w)

        def wait_fetch(j, gbuf, wbuf, sem_g, sem_w):
            pltpu.make_async_copy(h_hbm.at[src_v.at[j]], gbuf, sem_g).wait()
            pltpu.make_async_copy(
                w_hbm.at[pl.ds(ebase + j * B, B), pl.ds(c * HH, HH)],
                wbuf, sem_w).wait()

        def mul(gbuf, wbuf):
            def mrow(r4, inner):
                for dr in range(4):
                    for k in range(HH // 16):
                        sl = pl.ds(k * 16, 16)
                        r = r4 * 4 + dr
                        gbuf[r, sl] = gbuf[r, sl] * wbuf[r, sl]
                return inner
            lax.fori_loop(0, B // 4, mrow, 0)

        def scat(j, gbuf, sem_s):
            pltpu.async_copy(gbuf, acc.at[dst_v.at[j]], sem_s, add=True)

        def wait_scat(j, gbuf, sem_s):
            pltpu.make_async_copy(gbuf, acc.at[dst_v.at[j]], sem_s).wait()

        fetch(0, gbuf_a, wbuf_a, sem_ga, sem_wa)

        def body(j2, carry):
            a = j2 * 2
            b = a + 1
            wait_fetch(a, gbuf_a, wbuf_a, sem_ga, sem_wa)

            @pl.when(j2 > 0)
            def _():
                wait_scat(b, gbuf_b, sem_sb)

            fetch(b, gbuf_b, wbuf_b, sem_gb, sem_wb)
            mul(gbuf_a, wbuf_a)
            scat(a, gbuf_a, sem_sa)
            wait_fetch(b, gbuf_b, wbuf_b, sem_gb, sem_wb)
            wait_scat(a, gbuf_a, sem_sa)

            @pl.when(j2 < nb // 2 - 1)
            def _():
                fetch(a + 2, gbuf_a, wbuf_a, sem_ga, sem_wa)

            mul(gbuf_b, wbuf_b)
            scat(b, gbuf_b, sem_sb)
            return carry

        lax.fori_loop(0, nb // 2, body, 0)
        wait_scat(nb - 1, gbuf_b, sem_sb)

        plsc.subcore_barrier()
        pltpu.sync_copy(acc.at[pl.ds(s * RPT, RPT)],
                        out_hbm.at[c, pl.ds(s * RPT, RPT)])

    mesh = plsc.VectorSubcoreMesh(
        core_axis_name="c", subcore_axis_name="s",
        num_cores=NC, num_subcores=NS)
    return pl.kernel(
        _sc_msgpass,
        out_type=jax.ShapeDtypeStruct((NC, NP, HH), jnp.float32),
        mesh=mesh,
        scratch_types=[
            pltpu.VMEM((nb, B), jnp.int32),
            pltpu.VMEM((nb, B), jnp.int32),
            pltpu.VMEM((B, HH), jnp.float32),
            pltpu.VMEM((B, HH), jnp.float32),
            pltpu.VMEM((B, HH), jnp.float32),
            pltpu.VMEM((B, HH), jnp.float32),
            pltpu.VMEM((ZR, HH), jnp.float32),
            pltpu.VMEM_SHARED((NP, HH), jnp.float32),
            pltpu.SemaphoreType.DMA,
            pltpu.SemaphoreType.DMA,
            pltpu.SemaphoreType.DMA,
            pltpu.SemaphoreType.DMA,
            pltpu.SemaphoreType.DMA,
            pltpu.SemaphoreType.DMA,
        ],
        compiler_params=pltpu.CompilerParams(use_tc_tiling_on_sc=False),
    )


_BLK0 = 63
_BLK1 = (E // _BE) - _BLK0
_E0 = _BLK0 * _BE
_NB0 = _E0 // NS // B
_NB1 = (E - _E0) // NS // B


def kernel(x, edge_index, edge_length, edge_attr,
           nn0_w, nn0_b, nn2_w, nn2_b, lin1_w, lin2_w, lin2_b, lin_w, lin_b):
    ei = edge_index.astype(jnp.int32)
    src0 = ei[0, :_E0].reshape(NS, _NB0, B)
    dst0 = ei[1, :_E0].reshape(NS, _NB0, B)
    src1 = ei[0, _E0:].reshape(NS, _NB1, B)
    dst1 = ei[1, _E0:].reshape(NS, _NB1, B)
    el128 = edge_length.reshape(E // _BE, _BE // 128, 128)
    eat = edge_attr.T
    b0 = nn0_b.reshape(1, H)
    b2 = nn2_b.reshape(1, H)

    hcat = _lin1(x, lin1_w.T)
    w0 = _edge_filter(eat, el128, nn0_w.T, b0, nn2_w.T, b2, _BLK0, 0)
    w1 = _edge_filter(eat, el128, nn0_w.T, b0, nn2_w.T, b2, _BLK1, _BLK0)
    p0 = _sc_msgpass_call(_NB0, 0)(hcat, w0, src0, dst0)
    p1 = _sc_msgpass_call(_NB1, 0)(hcat, w1, src1, dst1)
    return _tail(p0, p1, lin2_w.T, lin2_b.reshape(1, H),
                 lin_w.T, lin_b.reshape(1, H))


# speedup vs baseline: 6.2485x; 1.2625x over previous
import functools

import jax
import jax.numpy as jnp
from jax import lax
from jax.experimental import pallas as pl
from jax.experimental.pallas import tpu as pltpu
from jax.experimental.pallas import tpu_sc as plsc

N, E, H, G = 10000, 320000, 128, 16
NP = 10240
HH = H // 2
NC, NS = 2, 16
EPW = E // NS
B = 80
NB = EPW // B
RPT = NP // NS
ZR = 128

_LOG2 = 0.6931471805599453
_PI = 3.141592653589793
_CUT = 10.0


def _ssp(v):
    return jnp.maximum(v, 0.0) + jnp.log1p(jnp.exp(-jnp.abs(v))) - _LOG2


_BE = 2560


def _filter_body(eat, el, w0, b0, w2, b2, out):
    a = eat[...]
    z = lax.dot_general(a, w0[...], (((0,), (0,)), ((), ())),
                        preferred_element_type=jnp.float32) + b0[...]
    w = jnp.dot(_ssp(z), w2[...], preferred_element_type=jnp.float32) + b2[...]
    l = el[0]
    env = 0.5 * (jnp.cos(l * (_PI / _CUT)) + 1.0)
    env = jnp.where((l <= _CUT) & (l >= 0.0), env, 0.0)
    w3 = w.reshape(_BE // 128, 128, H) * env[:, :, None]
    out[...] = w3.reshape(_BE, H)


def _edge_filter(edge_attr_t, el128, w0t, b0, w2t, b2, nblk, blk_off):
    be = _BE
    return pl.pallas_call(
        _filter_body,
        grid=(nblk,),
        in_specs=[
            pl.BlockSpec((G, be), lambda i: (0, blk_off + i)),
            pl.BlockSpec((1, be // 128, 128), lambda i: (blk_off + i, 0, 0)),
            pl.BlockSpec((G, H), lambda i: (0, 0)),
            pl.BlockSpec((1, H), lambda i: (0, 0)),
            pl.BlockSpec((H, H), lambda i: (0, 0)),
            pl.BlockSpec((1, H), lambda i: (0, 0)),
        ],
        out_specs=pl.BlockSpec((be, H), lambda i: (i, 0)),
        out_shape=jax.ShapeDtypeStruct((nblk * be, H), jnp.float32),
    )(edge_attr_t, el128, w0t, b0, w2t, b2)


def _lin1_body(xr, wr, out):
    h = jnp.dot(xr[...], wr[...], preferred_element_type=jnp.float32)
    out[pl.ds(0, N), :] = h[:, :HH]
    out[pl.ds(NP, N), :] = h[:, HH:]


def _lin1(x, w1t):
    return pl.pallas_call(
        _lin1_body,
        out_shape=jax.ShapeDtypeStruct((2 * NP, HH), jnp.float32),
    )(x, w1t)


def _tail_body(p0, p1, w2l, b2l, wl, bl, out):
    a = p0[...]
    b = p1[...]
    r = jnp.concatenate([a[0] + b[0], a[1] + b[1]], axis=1)
    t = _ssp(jnp.dot(r, w2l[...], preferred_element_type=jnp.float32) + b2l[...])
    out[...] = jnp.dot(t, wl[...], preferred_element_type=jnp.float32) + bl[...]


def _tail(part0, part1, w2lt, b2l, wlt, bl):
    bn = 2000
    return pl.pallas_call(
        _tail_body,
        grid=(N // bn,),
        in_specs=[
            pl.BlockSpec((NC, bn, HH), lambda i: (0, i, 0)),
            pl.BlockSpec((NC, bn, HH), lambda i: (0, i, 0)),
            pl.BlockSpec((H, H), lambda i: (0, 0)),
            pl.BlockSpec((1, H), lambda i: (0, 0)),
            pl.BlockSpec((H, H), lambda i: (0, 0)),
            pl.BlockSpec((1, H), lambda i: (0, 0)),
        ],
        out_specs=pl.BlockSpec((bn, H), lambda i: (i, 0)),
        out_shape=jax.ShapeDtypeStruct((N, H), jnp.float32),
    )(part0, part1, w2lt, b2l, wlt, bl)


@functools.cache
def _sc_msgpass_call(nb, chunk_base):
    epw = nb * B

    def _sc_msgpass(h_hbm, w_hbm, src_hbm, dst_hbm, out_hbm,
                    src_v, dst_v, gbuf_a, gbuf_b, gbuf_c, gbuf_d,
                    wbuf_a, wbuf_b, wbuf_c, wbuf_d, zbuf, acc,
                    sem_ga, sem_gb, sem_gc, sem_gd,
                    sem_wa, sem_wb, sem_wc, sem_wd,
                    sem_sa, sem_sb, sem_sc, sem_sd):
        c = lax.axis_index("c")
        s = lax.axis_index("s")

        pltpu.sync_copy(src_hbm.at[s], src_v)
        pltpu.sync_copy(dst_hbm.at[s], dst_v)

        coff = jnp.full((16,), c * NP, jnp.int32)

        def orow(r, carry):
            for k in range(B // 16):
                sl = pl.ds(k * 16, 16)
                src_v[r, sl] = src_v[r, sl] + coff
            return carry

        lax.fori_loop(0, nb, orow, 0)

        zeros = jnp.zeros((16,), jnp.float32)

        def zrow(r, carry):
            for k in range(HH // 16):
                zbuf[r, pl.ds(k * 16, 16)] = zeros
            return carry

        lax.fori_loop(0, ZR, zrow, 0)

        def zcopy(t, carry):
            pltpu.sync_copy(zbuf, acc.at[pl.ds(s * RPT + t * ZR, ZR)])
            return carry

        lax.fori_loop(0, RPT // ZR, zcopy, 0)
        plsc.subcore_barrier()

        ebase = chunk_base + s * epw

        def fetch(j, gbuf, wbuf, sem_g, sem_w):
            pltpu.async_copy(h_hbm.at[src_v.at[j]], gbuf, sem_g)
            pltpu.async_copy(
                w_hbm.at[pl.ds(ebase + j * B, B), pl.ds(c * HH, HH)],
                wbuf, sem_w)

        def wait_fetch(j, gbuf, wbuf, sem_g, sem_w):
            pltpu.make_async_copy(h_hbm.at[src_v.at[j]], gbuf, sem_g).wait()
            pltpu.make_async_copy(
                w_hbm.at[pl.ds(ebase + j * B, B), pl.ds(c * HH, HH)],
                wbuf, sem_w).wait()

        def mul(gbuf, wbuf):
            def mrow(r4, inner):
                for dr in range(4):
                    for k in range(HH // 16):
                        sl = pl.ds(k * 16, 16)
                        r = r4 * 4 + dr
                        gbuf[r, sl] = gbuf[r, sl] * wbuf[r, sl]
                return inner
            lax.fori_loop(0, B // 4, mrow, 0)

        def scat(j, gbuf, sem_s):
            pltpu.async_copy(gbuf, acc.at[dst_v.at[j]], sem_s, add=True)

        def wait_scat(j, gbuf, sem_s):
            pltpu.make_async_copy(gbuf, acc.at[dst_v.at[j]], sem_s).wait()

        K = 4
        gbufs = [gbuf_a, gbuf_b, gbuf_c, gbuf_d]
        wbufs = [wbuf_a, wbuf_b, wbuf_c, wbuf_d]
        sgs = [sem_ga, sem_gb, sem_gc, sem_gd]
        sws = [sem_wa, sem_wb, sem_wc, sem_wd]
        sss = [sem_sa, sem_sb, sem_sc, sem_sd]

        for t in range(K - 1):
            fetch(t, gbufs[t], wbufs[t], sgs[t], sws[t])

        ngrp = nb // K
        rem = nb - ngrp * K

        def step(j, t, drain):
            wait_fetch(j, gbufs[t], wbufs[t], sgs[t], sws[t])
            mul(gbufs[t], wbufs[t])
            scat(j, gbufs[t], sss[t])
            tf = (t - 1) % K
            jf = j + K - 1

            @pl.when(jf < nb)
            def _():
                if drain:
                    wait_scat(j - 1, gbufs[tf], sss[tf])
                fetch(jf, gbufs[tf], wbufs[tf], sgs[tf], sws[tf])

        for t in range(K):
            step(t, t, t > 0)

        def body(g, carry):
            j0 = g * K
            for t in range(K):
                step(j0 + t, t, True)
            return carry

        lax.fori_loop(1, ngrp, body, 0)
        for r in range(rem):
            step(ngrp * K + r, r, True)
        for r in range(K):
            j = nb - K + r
            wait_scat(j, gbufs[(nb - K + r) % K], sss[(nb - K + r) % K])

        plsc.subcore_barrier()
        pltpu.sync_copy(acc.at[pl.ds(s * RPT, RPT)],
                        out_hbm.at[c, pl.ds(s * RPT, RPT)])

    mesh = plsc.VectorSubcoreMesh(
        core_axis_name="c", subcore_axis_name="s",
        num_cores=NC, num_subcores=NS)
    return pl.kernel(
        _sc_msgpass,
        out_type=jax.ShapeDtypeStruct((NC, NP, HH), jnp.float32),
        mesh=mesh,
        scratch_types=[
            pltpu.VMEM((nb, B), jnp.int32),
            pltpu.VMEM((nb, B), jnp.int32),
            pltpu.VMEM((B, HH), jnp.float32),
            pltpu.VMEM((B, HH), jnp.float32),
            pltpu.VMEM((B, HH), jnp.float32),
            pltpu.VMEM((B, HH), jnp.float32),
            pltpu.VMEM((B, HH), jnp.float32),
            pltpu.VMEM((B, HH), jnp.float32),
            pltpu.VMEM((B, HH), jnp.float32),
            pltpu.VMEM((B, HH), jnp.float32),
            pltpu.VMEM((ZR, HH), jnp.float32),
            pltpu.VMEM_SHARED((NP, HH), jnp.float32),
        ] + [pltpu.SemaphoreType.DMA] * 12,
        compiler_params=pltpu.CompilerParams(use_tc_tiling_on_sc=False),
    )


_BLK0 = 63
_BLK1 = (E // _BE) - _BLK0
_E0 = _BLK0 * _BE
_NB0 = _E0 // NS // B
_NB1 = (E - _E0) // NS // B


def kernel(x, edge_index, edge_length, edge_attr,
           nn0_w, nn0_b, nn2_w, nn2_b, lin1_w, lin2_w, lin2_b, lin_w, lin_b):
    ei = edge_index.astype(jnp.int32)
    src0 = ei[0, :_E0].reshape(NS, _NB0, B)
    dst0 = ei[1, :_E0].reshape(NS, _NB0, B)
    src1 = ei[0, _E0:].reshape(NS, _NB1, B)
    dst1 = ei[1, _E0:].reshape(NS, _NB1, B)
    el128 = edge_length.reshape(E // _BE, _BE // 128, 128)
    eat = edge_attr.T
    b0 = nn0_b.reshape(1, H)
    b2 = nn2_b.reshape(1, H)

    hcat = _lin1(x, lin1_w.T)
    w0 = _edge_filter(eat, el128, nn0_w.T, b0, nn2_w.T, b2, _BLK0, 0)
    w1 = _edge_filter(eat, el128, nn0_w.T, b0, nn2_w.T, b2, _BLK1, _BLK0)
    p0 = _sc_msgpass_call(_NB0, 0)(hcat, w0, src0, dst0)
    p1 = _sc_msgpass_call(_NB1, 0)(hcat, w1, src1, dst1)
    return _tail(p0, p1, lin2_w.T, lin2_b.reshape(1, H),
                 lin_w.T, lin_b.reshape(1, H))


# speedup vs baseline: 6.3714x; 1.0197x over previous
import functools

import jax
import jax.numpy as jnp
from jax import lax
from jax.experimental import pallas as pl
from jax.experimental.pallas import tpu as pltpu
from jax.experimental.pallas import tpu_sc as plsc

N, E, H, G = 10000, 320000, 128, 16
NP = 10240
HH = H // 2
NC, NS = 2, 16
EPW = E // NS
B = 80
NB = EPW // B
RPT = NP // NS
ZR = 128

_LOG2 = 0.6931471805599453
_PI = 3.141592653589793
_CUT = 10.0


def _ssp(v):
    return jnp.maximum(v, 0.0) + jnp.log1p(jnp.exp(-jnp.abs(v))) - _LOG2


_BE = 2560


def _filter_body(eat, el, w0, b0, w2, b2, out):
    a = eat[...]
    z = lax.dot_general(a, w0[...], (((0,), (0,)), ((), ())),
                        preferred_element_type=jnp.float32) + b0[...]
    w = jnp.dot(_ssp(z), w2[...], preferred_element_type=jnp.float32) + b2[...]
    l = el[0]
    env = 0.5 * (jnp.cos(l * (_PI / _CUT)) + 1.0)
    env = jnp.where((l <= _CUT) & (l >= 0.0), env, 0.0)
    w3 = w.reshape(_BE // 128, 128, H) * env[:, :, None]
    out[...] = w3.reshape(_BE, H)


def _edge_filter(edge_attr_t, el128, w0t, b0, w2t, b2, nblk, blk_off):
    be = _BE
    return pl.pallas_call(
        _filter_body,
        grid=(nblk,),
        in_specs=[
            pl.BlockSpec((G, be), lambda i: (0, blk_off + i)),
            pl.BlockSpec((1, be // 128, 128), lambda i: (blk_off + i, 0, 0)),
            pl.BlockSpec((G, H), lambda i: (0, 0)),
            pl.BlockSpec((1, H), lambda i: (0, 0)),
            pl.BlockSpec((H, H), lambda i: (0, 0)),
            pl.BlockSpec((1, H), lambda i: (0, 0)),
        ],
        out_specs=pl.BlockSpec((be, H), lambda i: (i, 0)),
        out_shape=jax.ShapeDtypeStruct((nblk * be, H), jnp.float32),
    )(edge_attr_t, el128, w0t, b0, w2t, b2)


def _lin1_body(xr, wr, out):
    h = jnp.dot(xr[...], wr[...], preferred_element_type=jnp.float32)
    out[pl.ds(0, N), :] = h[:, :HH]
    out[pl.ds(NP, N), :] = h[:, HH:]


def _lin1(x, w1t):
    return pl.pallas_call(
        _lin1_body,
        out_shape=jax.ShapeDtypeStruct((2 * NP, HH), jnp.float32),
    )(x, w1t)


def _tail_body(p0, p1, p2, p3, w2l, b2l, wl, bl, out):
    acc = p0[...] + p1[...] + p2[...] + p3[...]
    r = jnp.concatenate([acc[0, :, :HH], acc[1, :, :HH]], axis=1)
    t = _ssp(jnp.dot(r, w2l[...], preferred_element_type=jnp.float32) + b2l[...])
    out[...] = jnp.dot(t, wl[...], preferred_element_type=jnp.float32) + bl[...]


def _tail(parts, w2lt, b2l, wlt, bl):
    bn = 2000
    pspec = pl.BlockSpec((NC, bn, H), lambda i: (0, i, 0))
    return pl.pallas_call(
        _tail_body,
        grid=(N // bn,),
        in_specs=[
            pspec, pspec, pspec, pspec,
            pl.BlockSpec((H, H), lambda i: (0, 0)),
            pl.BlockSpec((1, H), lambda i: (0, 0)),
            pl.BlockSpec((H, H), lambda i: (0, 0)),
            pl.BlockSpec((1, H), lambda i: (0, 0)),
        ],
        out_specs=pl.BlockSpec((bn, H), lambda i: (i, 0)),
        out_shape=jax.ShapeDtypeStruct((N, H), jnp.float32),
    )(*parts, w2lt, b2l, wlt, bl)


@functools.cache
def _sc_msgpass_call(nb, chunk_base):
    epw = nb * B

    def _sc_msgpass(h_hbm, w_hbm, src_hbm, dst_hbm, out_hbm,
                    src_v, dst_v, gbuf_a, gbuf_b, gbuf_c, gbuf_d,
                    wbuf_a, wbuf_b, wbuf_c, wbuf_d, zbuf, acc,
                    sem_ga, sem_gb, sem_gc, sem_gd,
                    sem_wa, sem_wb, sem_wc, sem_wd,
                    sem_sa, sem_sb, sem_sc, sem_sd):
        c = lax.axis_index("c")
        s = lax.axis_index("s")

        pltpu.sync_copy(src_hbm.at[s], src_v)
        pltpu.sync_copy(dst_hbm.at[s], dst_v)

        coff = jnp.full((16,), c * NP, jnp.int32)

        def orow(r, carry):
            for k in range(B // 16):
                sl = pl.ds(k * 16, 16)
                src_v[r, sl] = src_v[r, sl] + coff
            return carry

        lax.fori_loop(0, nb, orow, 0)

        zeros = jnp.zeros((16,), jnp.float32)

        def zrow(r, carry):
            for k in range(HH // 16):
                zbuf[r, pl.ds(k * 16, 16)] = zeros
            return carry

        lax.fori_loop(0, ZR, zrow, 0)

        def zcopy(t, carry):
            pltpu.sync_copy(zbuf, acc.at[pl.ds(s * RPT + t * ZR, ZR)])
            return carry

        lax.fori_loop(0, RPT // ZR, zcopy, 0)
        plsc.subcore_barrier()

        ebase = chunk_base + s * epw

        def fetch(j, gbuf, wbuf, sem_g, sem_w):
            pltpu.async_copy(h_hbm.at[src_v.at[j]], gbuf, sem_g)
            pltpu.async_copy(
                w_hbm.at[pl.ds(ebase + j * B, B), pl.ds(c * HH, HH)],
                wbuf, sem_w)

        def wait_fetch(j, gbuf, wbuf, sem_g, sem_w):
            pltpu.make_async_copy(h_hbm.at[src_v.at[j]], gbuf, sem_g).wait()
            pltpu.make_async_copy(
                w_hbm.at[pl.ds(ebase + j * B, B), pl.ds(c * HH, HH)],
                wbuf, sem_w).wait()

        def mul(gbuf, wbuf):
            def mrow(r4, inner):
                for dr in range(4):
                    for k in range(HH // 16):
                        sl = pl.ds(k * 16, 16)
                        r = r4 * 4 + dr
                        gbuf[r, sl] = gbuf[r, sl] * wbuf[r, sl]
                return inner
            lax.fori_loop(0, B // 4, mrow, 0)

        def scat(j, gbuf, sem_s):
            pltpu.async_copy(gbuf, acc.at[dst_v.at[j]], sem_s, add=True)

        def wait_scat(j, gbuf, sem_s):
            pltpu.make_async_copy(gbuf, acc.at[dst_v.at[j]], sem_s).wait()

        K = 4
        gbufs = [gbuf_a, gbuf_b, gbuf_c, gbuf_d]
        wbufs = [wbuf_a, wbuf_b, wbuf_c, wbuf_d]
        sgs = [sem_ga, sem_gb, sem_gc, sem_gd]
        sws = [sem_wa, sem_wb, sem_wc, sem_wd]
        sss = [sem_sa, sem_sb, sem_sc, sem_sd]

        for t in range(K - 1):
            fetch(t, gbufs[t], wbufs[t], sgs[t], sws[t])

        ngrp = nb // K
        rem = nb - ngrp * K

        def step(j, t, drain):
            wait_fetch(j, gbufs[t], wbufs[t], sgs[t], sws[t])
            mul(gbufs[t], wbufs[t])
            scat(j, gbufs[t], sss[t])
            tf = (t - 1) % K
            jf = j + K - 1

            @pl.when(jf < nb)
            def _():
                if drain:
                    wait_scat(j - 1, gbufs[tf], sss[tf])
                fetch(jf, gbufs[tf], wbufs[tf], sgs[tf], sws[tf])

        for t in range(K):
            step(t, t, t > 0)

        def body(g, carry):
            j0 = g * K
            for t in range(K):
                step(j0 + t, t, True)
            return carry

        lax.fori_loop(1, ngrp, body, 0)
        for r in range(rem):
            step(ngrp * K + r, r, True)
        for r in range(K):
            j = nb - K + r
            wait_scat(j, gbufs[(nb - K + r) % K], sss[(nb - K + r) % K])

        plsc.subcore_barrier()
        pltpu.sync_copy(acc.at[pl.ds(s * RPT, RPT)],
                        out_hbm.at[c, pl.ds(s * RPT, RPT), pl.ds(0, HH)])

    mesh = plsc.VectorSubcoreMesh(
        core_axis_name="c", subcore_axis_name="s",
        num_cores=NC, num_subcores=NS)
    return pl.kernel(
        _sc_msgpass,
        out_type=jax.ShapeDtypeStruct((NC, NP, H), jnp.float32),
        mesh=mesh,
        scratch_types=[
            pltpu.VMEM((nb, B), jnp.int32),
            pltpu.VMEM((nb, B), jnp.int32),
            pltpu.VMEM((B, HH), jnp.float32),
            pltpu.VMEM((B, HH), jnp.float32),
            pltpu.VMEM((B, HH), jnp.float32),
            pltpu.VMEM((B, HH), jnp.float32),
            pltpu.VMEM((B, HH), jnp.float32),
            pltpu.VMEM((B, HH), jnp.float32),
            pltpu.VMEM((B, HH), jnp.float32),
            pltpu.VMEM((B, HH), jnp.float32),
            pltpu.VMEM((ZR, HH), jnp.float32),
            pltpu.VMEM_SHARED((NP, HH), jnp.float32),
        ] + [pltpu.SemaphoreType.DMA] * 12,
        compiler_params=pltpu.CompilerParams(use_tc_tiling_on_sc=False),
    )


_CHUNKS = (20, 28, 36, 41)


def kernel(x, edge_index, edge_length, edge_attr,
           nn0_w, nn0_b, nn2_w, nn2_b, lin1_w, lin2_w, lin2_b, lin_w, lin_b):
    ei = edge_index.astype(jnp.int32)
    el128 = edge_length.reshape(E // _BE, _BE // 128, 128)
    eat = edge_attr.T
    w0t = nn0_w.T
    b0 = nn0_b.reshape(1, H)
    w2t = nn2_w.T
    b2 = nn2_b.reshape(1, H)

    hcat = _lin1(x, lin1_w.T)
    parts = []
    blk_off = 0
    for nblk in _CHUNKS:
        e0 = blk_off * _BE
        e1 = e0 + nblk * _BE
        nb = (e1 - e0) // NS // B
        src3 = ei[0, e0:e1].reshape(NS, nb, B)
        dst3 = ei[1, e0:e1].reshape(NS, nb, B)
        w = _edge_filter(eat, el128, w0t, b0, w2t, b2, nblk, blk_off)
        parts.append(_sc_msgpass_call(nb, 0)(hcat, w, src3, dst3))
        blk_off += nblk
    return _tail(parts, lin2_w.T, lin2_b.reshape(1, H),
                 lin_w.T, lin_b.reshape(1, H))


# speedup vs baseline: 6.8269x; 1.0715x over previous
import functools

import jax
import jax.numpy as jnp
from jax import lax
from jax.experimental import pallas as pl
from jax.experimental.pallas import tpu as pltpu
from jax.experimental.pallas import tpu_sc as plsc

N, E, H, G = 10000, 320000, 128, 16
NP = 10240
HH = H // 2
NC, NS = 2, 16
EPW = E // NS
B = 80
NB = EPW // B
RPT = NP // NS
ZR = 128

_LOG2 = 0.6931471805599453
_PI = 3.141592653589793
_CUT = 10.0


def _ssp(v):
    return jnp.maximum(v, 0.0) + jnp.log1p(jnp.exp(-jnp.abs(v))) - _LOG2


_BE = 2560


def _filter_body(eat, el, w0, b0, w2, b2, out):
    a = eat[...]
    z = lax.dot_general(a, w0[...], (((0,), (0,)), ((), ())),
                        preferred_element_type=jnp.float32) + b0[...]
    w = jnp.dot(_ssp(z), w2[...], preferred_element_type=jnp.float32) + b2[...]
    l = el[0]
    env = 0.5 * (jnp.cos(l * (_PI / _CUT)) + 1.0)
    env = jnp.where((l <= _CUT) & (l >= 0.0), env, 0.0)
    w3 = w.reshape(_BE // 128, 128, H) * env[:, :, None]
    out[...] = w3.reshape(_BE, H)


def _edge_filter(edge_attr_t, el128, w0t, b0, w2t, b2, nblk, blk_off):
    be = _BE
    return pl.pallas_call(
        _filter_body,
        grid=(nblk,),
        in_specs=[
            pl.BlockSpec((G, be), lambda i: (0, blk_off + i)),
            pl.BlockSpec((1, be // 128, 128), lambda i: (blk_off + i, 0, 0)),
            pl.BlockSpec((G, H), lambda i: (0, 0)),
            pl.BlockSpec((1, H), lambda i: (0, 0)),
            pl.BlockSpec((H, H), lambda i: (0, 0)),
            pl.BlockSpec((1, H), lambda i: (0, 0)),
        ],
        out_specs=pl.BlockSpec((be, H), lambda i: (i, 0)),
        out_shape=jax.ShapeDtypeStruct((nblk * be, H), jnp.float32),
    )(edge_attr_t, el128, w0t, b0, w2t, b2)


def _lin1_body(xr, wr, out):
    h = jnp.dot(xr[...], wr[...], preferred_element_type=jnp.float32)
    out[pl.ds(0, N), :] = h[:, :HH]
    out[pl.ds(NP, N), :] = h[:, HH:]


def _lin1(x, w1t):
    return pl.pallas_call(
        _lin1_body,
        out_shape=jax.ShapeDtypeStruct((2 * NP, HH), jnp.float32),
    )(x, w1t)


def _tail_body(p0, p1, p2, p3, w2l, b2l, wl, bl, out):
    acc = p0[...] + p1[...] + p2[...] + p3[...]
    r = jnp.concatenate([acc[0, :, :HH], acc[1, :, :HH]], axis=1)
    t = _ssp(jnp.dot(r, w2l[...], preferred_element_type=jnp.float32) + b2l[...])
    out[...] = jnp.dot(t, wl[...], preferred_element_type=jnp.float32) + bl[...]


def _tail(parts, w2lt, b2l, wlt, bl):
    bn = 2000
    pspec = pl.BlockSpec((NC, bn, H), lambda i: (0, i, 0))
    return pl.pallas_call(
        _tail_body,
        grid=(N // bn,),
        in_specs=[
            pspec, pspec, pspec, pspec,
            pl.BlockSpec((H, H), lambda i: (0, 0)),
            pl.BlockSpec((1, H), lambda i: (0, 0)),
            pl.BlockSpec((H, H), lambda i: (0, 0)),
            pl.BlockSpec((1, H), lambda i: (0, 0)),
        ],
        out_specs=pl.BlockSpec((bn, H), lambda i: (i, 0)),
        out_shape=jax.ShapeDtypeStruct((N, H), jnp.float32),
    )(*parts, w2lt, b2l, wlt, bl)


@functools.cache
def _sc_msgpass_call(nb, chunk_base):
    epw = nb * B

    def _sc_msgpass(h_hbm, w_hbm, src_hbm, dst_hbm, out_hbm,
                    src_v, dst_v, gbuf_a, gbuf_b, gbuf_c, gbuf_d,
                    wbuf_a, wbuf_b, wbuf_c, wbuf_d, zbuf, acc,
                    sem_ga, sem_gb, sem_gc, sem_gd,
                    sem_wa, sem_wb, sem_wc, sem_wd,
                    sem_sa, sem_sb, sem_sc, sem_sd):
        c = lax.axis_index("c")
        s = lax.axis_index("s")

        pltpu.sync_copy(src_hbm.at[s], src_v)
        pltpu.sync_copy(dst_hbm.at[s], dst_v)

        coff = jnp.full((16,), c * NP, jnp.int32)

        def orow(r, carry):
            for k in range(B // 16):
                sl = pl.ds(k * 16, 16)
                src_v[r, sl] = src_v[r, sl] + coff
            return carry

        lax.fori_loop(0, nb, orow, 0)

        zeros = jnp.zeros((16,), jnp.float32)

        def zrow(r, carry):
            for k in range(HH // 16):
                zbuf[r, pl.ds(k * 16, 16)] = zeros
            return carry

        lax.fori_loop(0, ZR, zrow, 0)

        def zcopy(t, carry):
            pltpu.sync_copy(zbuf, acc.at[pl.ds(s * RPT + t * ZR, ZR)])
            return carry

        lax.fori_loop(0, RPT // ZR, zcopy, 0)
        plsc.subcore_barrier()

        ebase = chunk_base + s * epw

        def fetch(j, gbuf, wbuf, sem_g, sem_w):
            pltpu.async_copy(h_hbm.at[src_v.at[j]], gbuf, sem_g)
            pltpu.async_copy(
                w_hbm.at[pl.ds(ebase + j * B, B), pl.ds(c * HH, HH)],
                wbuf, sem_w)

        def wait_fetch(j, gbuf, wbuf, sem_g, sem_w):
            pltpu.make_async_copy(h_hbm.at[src_v.at[j]], gbuf, sem_g).wait()
            pltpu.make_async_copy(
                w_hbm.at[pl.ds(ebase + j * B, B), pl.ds(c * HH, HH)],
                wbuf, sem_w).wait()

        def mul(gbuf, wbuf):
            def mrow(r4, inner):
                for dr in range(4):
                    for k in range(HH // 16):
                        sl = pl.ds(k * 16, 16)
                        r = r4 * 4 + dr
                        gbuf[r, sl] = gbuf[r, sl] * wbuf[r, sl]
                return inner
            lax.fori_loop(0, B // 4, mrow, 0)

        def scat(j, gbuf, sem_s):
            pltpu.async_copy(gbuf, acc.at[dst_v.at[j]], sem_s, add=True)

        def wait_scat(j, gbuf, sem_s):
            pltpu.make_async_copy(gbuf, acc.at[dst_v.at[j]], sem_s).wait()

        K = 4
        gbufs = [gbuf_a, gbuf_b, gbuf_c, gbuf_d]
        wbufs = [wbuf_a, wbuf_b, wbuf_c, wbuf_d]
        sgs = [sem_ga, sem_gb, sem_gc, sem_gd]
        sws = [sem_wa, sem_wb, sem_wc, sem_wd]
        sss = [sem_sa, sem_sb, sem_sc, sem_sd]

        for t in range(K - 1):
            fetch(t, gbufs[t], wbufs[t], sgs[t], sws[t])

        ngrp = nb // K
        rem = nb - ngrp * K

        def step(j, t, drain):
            wait_fetch(j, gbufs[t], wbufs[t], sgs[t], sws[t])
            mul(gbufs[t], wbufs[t])
            scat(j, gbufs[t], sss[t])
            tf = (t - 1) % K
            jf = j + K - 1

            @pl.when(jf < nb)
            def _():
                if drain:
                    wait_scat(j - 1, gbufs[tf], sss[tf])
                fetch(jf, gbufs[tf], wbufs[tf], sgs[tf], sws[tf])

        for t in range(K):
            step(t, t, t > 0)

        def body(g, carry):
            j0 = g * K
            for t in range(K):
                step(j0 + t, t, True)
            return carry

        lax.fori_loop(1, ngrp, body, 0)
        for r in range(rem):
            step(ngrp * K + r, r, True)
        for r in range(K):
            j = nb - K + r
            wait_scat(j, gbufs[(nb - K + r) % K], sss[(nb - K + r) % K])

        plsc.subcore_barrier()
        pltpu.sync_copy(acc.at[pl.ds(s * RPT, RPT)],
                        out_hbm.at[c, pl.ds(s * RPT, RPT), pl.ds(0, HH)])

    mesh = plsc.VectorSubcoreMesh(
        core_axis_name="c", subcore_axis_name="s",
        num_cores=NC, num_subcores=NS)
    return pl.kernel(
        _sc_msgpass,
        out_type=jax.ShapeDtypeStruct((NC, NP, H), jnp.float32),
        mesh=mesh,
        scratch_types=[
            pltpu.VMEM((nb, B), jnp.int32),
            pltpu.VMEM((nb, B), jnp.int32),
            pltpu.VMEM((B, HH), jnp.float32),
            pltpu.VMEM((B, HH), jnp.float32),
            pltpu.VMEM((B, HH), jnp.float32),
            pltpu.VMEM((B, HH), jnp.float32),
            pltpu.VMEM((B, HH), jnp.float32),
            pltpu.VMEM((B, HH), jnp.float32),
            pltpu.VMEM((B, HH), jnp.float32),
            pltpu.VMEM((B, HH), jnp.float32),
            pltpu.VMEM((ZR, HH), jnp.float32),
            pltpu.VMEM_SHARED((NP, HH), jnp.float32),
        ] + [pltpu.SemaphoreType.DMA] * 12,
        compiler_params=pltpu.CompilerParams(use_tc_tiling_on_sc=False),
    )


_CHUNKS = (20, 28, 36, 41)


def kernel(x, edge_index, edge_length, edge_attr,
           nn0_w, nn0_b, nn2_w, nn2_b, lin1_w, lin2_w, lin2_b, lin_w, lin_b):
    ei = edge_index.astype(jnp.int32)
    el128 = edge_length.reshape(E // _BE, _BE // 128, 128)
    eat = edge_attr.T
    w0t = nn0_w.T
    b0 = nn0_b.reshape(1, H)
    w2t = nn2_w.T
    b2 = nn2_b.reshape(1, H)

    hcat = _lin1(x, lin1_w.T)
    parts = []
    blk_off = 0
    w = None
    for nblk in _CHUNKS:
        e0 = blk_off * _BE
        e1 = e0 + nblk * _BE
        nb = (e1 - e0) // NS // B
        src3 = ei[0, e0:e1].reshape(NS, nb, B)
        dst3 = ei[1, e0:e1].reshape(NS, nb, B)
        b0c = b0 if w is None else b0 + w[0, 0] * 0.0
        w = _edge_filter(eat, el128, w0t, b0c, w2t, b2, nblk, blk_off)
        parts.append(_sc_msgpass_call(nb, 0)(hcat, w, src3, dst3))
        blk_off += nblk
    return _tail(parts, lin2_w.T, lin2_b.reshape(1, H),
                 lin_w.T, lin_b.reshape(1, H))
